# baseline (speedup 1.0000x reference)
import jax
import jax.numpy as jnp
import numpy as np
from jax.experimental import pallas as pl
from jax.experimental.pallas import tpu as pltpu

N_EXPERTS_ = 8
TOP_K_ = 2
RANK_ = 32
ER_ = N_EXPERTS_ * RANK_
SCALING_ = 16.0 / np.sqrt(RANK_)

BM = 1024
BN = 512


def _moe_kernel(idx_ref, x_ref, wb_ref, a_ref, b_ref, out_ref, xh_scratch):
    nj = pl.program_id(1)

    @pl.when(nj == 0)
    def _():
        xb = x_ref[...].astype(jnp.bfloat16)
        xh_scratch[:, :x_ref.shape[1]] = xb
        h = jax.lax.dot_general(
            xb, a_ref[...],
            (((1,), (1,)), ((), ())),
            preferred_element_type=jnp.float32)
        idx = idx_ref[...]
        eid = jax.lax.broadcasted_iota(jnp.int32, (BM, ER_), 1) // RANK_
        mask = (eid == idx[:, 0:1]) | (eid == idx[:, 1:2])
        xh_scratch[:, x_ref.shape[1]:] = jnp.where(
            mask, h * SCALING_, 0.0).astype(jnp.bfloat16)

    acc = jax.lax.dot_general(
        xh_scratch[...], wb_ref[...],
        (((1,), (1,)), ((), ())),
        preferred_element_type=jnp.float32)
    out_ref[...] = acc + b_ref[...]


@jax.jit
def kernel(x, expert_indices, W, b, lora_A, lora_B):
    x_shape = x.shape
    x2 = x.reshape(-1, x_shape[-1])
    n, d = x2.shape
    idx = expert_indices.reshape(-1, expert_indices.shape[-1]).astype(jnp.int32)
    a_all = lora_A.reshape(ER_, d).astype(jnp.bfloat16)
    b_flat = lora_B.transpose(0, 2, 1).reshape(ER_, d).astype(jnp.bfloat16)
    wb = jnp.concatenate([W.astype(jnp.bfloat16), b_flat.T], axis=1)
    b2 = b.reshape(1, d)

    grid = (n // BM, d // BN)
    out = pl.pallas_call(
        _moe_kernel,
        grid=grid,
        in_specs=[
            pl.BlockSpec((BM, TOP_K_), lambda i, j: (i, 0)),
            pl.BlockSpec((BM, d), lambda i, j: (i, 0)),
            pl.BlockSpec((BN, d + ER_), lambda i, j: (j, 0)),
            pl.BlockSpec((ER_, d), lambda i, j: (0, 0)),
            pl.BlockSpec((1, BN), lambda i, j: (0, j)),
        ],
        out_specs=pl.BlockSpec((BM, BN), lambda i, j: (i, j)),
        out_shape=jax.ShapeDtypeStruct((n, d), jnp.float32),
        scratch_shapes=[pltpu.VMEM((BM, d + ER_), jnp.bfloat16)],
    )(idx, x2, wb, a_all, b2)
    return out.reshape(x_shape[:-1] + (d,))


# speedup vs baseline: 1.0622x; 1.0622x over previous
import jax
import jax.numpy as jnp
import numpy as np
from jax.experimental import pallas as pl
from jax.experimental.pallas import tpu as pltpu

N_EXPERTS_ = 8
TOP_K_ = 2
RANK_ = 32
ER_ = N_EXPERTS_ * RANK_
SCALING_ = 16.0 / np.sqrt(RANK_)

BM = 1024
BN = 512


def _moe_kernel(idx_ref, x_ref, w_ref, a_ref, bflat_ref, b_ref, out_ref,
                h_scratch, xb_scratch):
    nj = pl.program_id(1)

    @pl.when(nj == 0)
    def _():
        xb = x_ref[...].astype(jnp.bfloat16)
        xb_scratch[...] = xb
        h = jax.lax.dot_general(
            xb, a_ref[...],
            (((1,), (1,)), ((), ())),
            preferred_element_type=jnp.float32)
        idx = idx_ref[...]
        eid = jax.lax.broadcasted_iota(jnp.int32, (BM, ER_), 1) // RANK_
        mask = (eid == idx[:, 0:1]) | (eid == idx[:, 1:2])
        h_scratch[...] = jnp.where(mask, h * SCALING_, 0.0).astype(jnp.bfloat16)

    @pl.when(nj > 0)
    def _():
        acc = jax.lax.dot_general(
            xb_scratch[...], w_ref[...],
            (((1,), (1,)), ((), ())),
            preferred_element_type=jnp.float32)
        acc += jnp.dot(h_scratch[...], bflat_ref[...],
                       preferred_element_type=jnp.float32)
        out_ref[...] = acc + b_ref[...]


@jax.jit
def kernel(x, expert_indices, W, b, lora_A, lora_B):
    x_shape = x.shape
    x2 = x.reshape(-1, x_shape[-1])
    n, d = x2.shape
    idx = expert_indices.reshape(-1, expert_indices.shape[-1]).astype(jnp.int32)
    W = W.astype(jnp.bfloat16)
    a_all = lora_A.reshape(ER_, d).astype(jnp.bfloat16)
    b_flat = lora_B.transpose(0, 2, 1).reshape(ER_, d).astype(jnp.bfloat16)
    b2 = b.reshape(1, d)

    def jcol(j):
        return jnp.maximum(j - 1, 0)

    grid = (n // BM, d // BN + 1)
    out = pl.pallas_call(
        _moe_kernel,
        grid=grid,
        in_specs=[
            pl.BlockSpec((BM, TOP_K_), lambda i, j: (i, 0)),
            pl.BlockSpec((BM, d), lambda i, j: (i, 0)),
            pl.BlockSpec((BN, d), lambda i, j: (jcol(j), 0)),
            pl.BlockSpec((ER_, d), lambda i, j: (0, 0)),
            pl.BlockSpec((ER_, BN), lambda i, j: (0, jcol(j))),
            pl.BlockSpec((1, BN), lambda i, j: (0, jcol(j))),
        ],
        out_specs=pl.BlockSpec((BM, BN), lambda i, j: (i, jcol(j))),
        out_shape=jax.ShapeDtypeStruct((n, d), jnp.float32),
        scratch_shapes=[pltpu.VMEM((BM, ER_), jnp.bfloat16),
                        pltpu.VMEM((BM, d), jnp.bfloat16)],
    )(idx, x2, W, a_all, b_flat, b2)
    return out.reshape(x_shape[:-1] + (d,))


# speedup vs baseline: 1.0732x; 1.0104x over previous
import jax
import jax.numpy as jnp
import numpy as np
from jax.experimental import pallas as pl
from jax.experimental.pallas import tpu as pltpu

N_EXPERTS_ = 8
TOP_K_ = 2
RANK_ = 32
ER_ = N_EXPERTS_ * RANK_
SCALING_ = 16.0 / np.sqrt(RANK_)

BM = 1024
BN = 512


def _moe_kernel(idx_ref, x_ref, w_ref, a_ref, bflat_ref, b_ref, out_ref,
                h_scratch, xb_scratch):
    nj = pl.program_id(1)

    @pl.when(nj == 0)
    def _():
        xb = x_ref[...].astype(jnp.bfloat16)
        xb_scratch[...] = xb
        h = jax.lax.dot_general(
            xb, a_ref[...],
            (((1,), (1,)), ((), ())),
            preferred_element_type=jnp.float32)
        idx = idx_ref[...]
        eid = jax.lax.broadcasted_iota(jnp.int32, (BM, ER_), 1) // RANK_
        mask = (eid == idx[:, 0:1]) | (eid == idx[:, 1:2])
        h_scratch[...] = jnp.where(mask, h * SCALING_, 0.0).astype(jnp.bfloat16)

    acc = jax.lax.dot_general(
        xb_scratch[...], w_ref[...],
        (((1,), (1,)), ((), ())),
        preferred_element_type=jnp.float32)
    acc += jnp.dot(h_scratch[...], bflat_ref[...],
                   preferred_element_type=jnp.float32)
    out_ref[...] = acc + b_ref[...]


@jax.jit
def kernel(x, expert_indices, W, b, lora_A, lora_B):
    x_shape = x.shape
    x2 = x.reshape(-1, x_shape[-1])
    n, d = x2.shape
    idx = expert_indices.reshape(-1, expert_indices.shape[-1]).astype(jnp.int32)
    W = W.astype(jnp.bfloat16)
    a_all = lora_A.reshape(ER_, d).astype(jnp.bfloat16)
    b_flat = lora_B.transpose(0, 2, 1).reshape(ER_, d).astype(jnp.bfloat16)
    b2 = b.reshape(1, d)

    grid = (n // BM, d // BN)
    out = pl.pallas_call(
        _moe_kernel,
        grid=grid,
        in_specs=[
            pl.BlockSpec((BM, TOP_K_), lambda i, j: (i, 0)),
            pl.BlockSpec((BM, d), lambda i, j: (i, 0)),
            pl.BlockSpec((BN, d), lambda i, j: (j, 0)),
            pl.BlockSpec((ER_, d), lambda i, j: (0, 0)),
            pl.BlockSpec((ER_, BN), lambda i, j: (0, j)),
            pl.BlockSpec((1, BN), lambda i, j: (0, j)),
        ],
        out_specs=pl.BlockSpec((BM, BN), lambda i, j: (i, j)),
        out_shape=jax.ShapeDtypeStruct((n, d), jnp.float32),
        scratch_shapes=[pltpu.VMEM((BM, ER_), jnp.bfloat16),
                        pltpu.VMEM((BM, d), jnp.bfloat16)],
    )(idx, x2, W, a_all, b_flat, b2)
    return out.reshape(x_shape[:-1] + (d,))
